# baseline (speedup 1.0000x reference)
import functools

import jax
import jax.numpy as jnp
from jax import lax
from jax.experimental import pallas as pl
from jax.experimental.pallas import tpu as pltpu
from jax.experimental.pallas import tpu_sc as plsc

N = 50000
D = 128
E = 300000

NT = 16
CH = 128
CHUNKS = 152
EPT = CH * CHUNKS
E_PAD = EPT * NT
SEG = 6272
NPASS = 4
TRASH = SEG
ACC_ROWS = 6400
STRIPE = ACC_ROWS // NT
PAD_DST = 50168

CNT_BINS = 53248
EPT32 = E_PAD // 32


_SEG_SCRATCH = (
    [pltpu.VMEM((EPT,), jnp.int32)] * 2
    + [pltpu.VMEM((CH, D), jnp.float32)] * 2
    + [pltpu.VMEM((CH,), jnp.int32)] * 2
    + [pltpu.VMEM((CH,), jnp.int32)] * 2
    + [pltpu.VMEM_SHARED((ACC_ROWS, D), jnp.float32)]
    + [pltpu.SemaphoreType.DMA] * 4
)


@functools.partial(
    pl.kernel,
    out_type=jax.ShapeDtypeStruct((N, D), jnp.float32),
    mesh=plsc.VectorSubcoreMesh(core_axis_name="c", subcore_axis_name="s"),
    scratch_types=_SEG_SCRATCH,
)
def _segsum(table, src, dst, zrows, out,
            srcv, dstv, r0, r1, o0, o1, i0, i1, acc, g0, g1, s0, s1):
    rows = (r0, r1)
    offs = (o0, o1)
    gidx = (i0, i1)
    gsem = (g0, g1)
    ssem = (s0, s1)
    c = lax.axis_index("c")
    t = lax.axis_index("s")

    pltpu.sync_copy(src.at[pl.ds(t * EPT, EPT)], srcv)
    pltpu.sync_copy(dst.at[pl.ds(t * EPT, EPT)], dstv)

    def issue_gather(b):
        pltpu.async_copy(table.at[gidx[b]], rows[b], gsem[b])

    def wait_gather(b):
        pltpu.make_async_copy(table.at[gidx[b]], rows[b], gsem[b]).wait()

    def issue_scatter(b):
        pltpu.async_copy(rows[b], acc.at[offs[b]], ssem[b], add=True)

    def wait_scatter(b):
        pltpu.make_async_copy(rows[b], acc.at[offs[b]], ssem[b]).wait()

    for q in range(NPASS):
        lo = (NPASS * c + q) * SEG
        hi = jnp.minimum(lo + SEG, N)

        def compute_idx(ci, b, lo=lo, hi=hi):
            for g in range(CH // 16):
                d = dstv[pl.ds(ci * CH + g * 16, 16)]
                s_v = srcv[pl.ds(ci * CH + g * 16, 16)]
                inr = (d >= lo) & (d < hi)
                gidx[b][pl.ds(g * 16, 16)] = jnp.where(inr, s_v, 0)
                offs[b][pl.ds(g * 16, 16)] = jnp.where(inr, d - lo, TRASH)

        pltpu.sync_copy(zrows, acc.at[pl.ds(t * STRIPE, STRIPE)])
        plsc.subcore_barrier()

        compute_idx(0, 0)
        issue_gather(0)

        def step(k2, carry):
            for u in range(2):
                ci = 2 * k2 + u
                nb = 1 - u
                compute_idx(jnp.minimum(ci + 1, CHUNKS - 1), nb)
                issue_gather(nb)
                wait_gather(u)
                pltpu.async_copy(rows[u], acc.at[offs[u]], ssem[u], add=True).wait()
            return carry

        lax.fori_loop(0, CHUNKS // 2, step, 0)
        wait_gather(0)

        plsc.subcore_barrier()
        wb = pl.multiple_of(jnp.minimum(t * STRIPE, hi - lo - STRIPE), 8)
        pltpu.sync_copy(acc.at[pl.ds(wb, STRIPE)],
                        out.at[pl.ds(pl.multiple_of(lo + wb, 8), STRIPE)])
        plsc.subcore_barrier()


_CNT_SCRATCH = [
    pltpu.VMEM((CNT_BINS,), jnp.float32),
    pltpu.VMEM((EPT32,), jnp.int32),
]


@functools.partial(
    pl.kernel,
    out_type=jax.ShapeDtypeStruct((32, CNT_BINS), jnp.float32),
    mesh=plsc.VectorSubcoreMesh(core_axis_name="c", subcore_axis_name="s"),
    scratch_types=_CNT_SCRATCH,
)
def _hist32(dst, zcnt, out, hist, dstv):
    c = lax.axis_index("c")
    t = lax.axis_index("s")
    wid = c * NT + t
    pltpu.sync_copy(zcnt, hist)
    pltpu.sync_copy(dst.at[pl.ds(wid * EPT32, EPT32)], dstv)

    inc = jnp.where(lax.iota(jnp.int32, 16) == 0, 1.0, 0.0).astype(jnp.float32)

    def step(g, carry):
        dvec = dstv[pl.ds(g * 16, 16)]
        for l in range(16):
            dl = dvec[l]
            hist[pl.ds(dl, 16)] = hist[pl.ds(dl, 16)] + inc
        return carry

    lax.fori_loop(0, EPT32 // 16, step, 0)
    pltpu.sync_copy(hist, out.at[wid])


def _invsum_body(h, inv):
    inv[...] = 1.0 / jnp.maximum(jnp.sum(h[...], axis=0), 1.0)[None, :]


_invsum = pl.pallas_call(
    _invsum_body,
    grid=(1,),
    in_specs=[pl.BlockSpec((32, CNT_BINS), lambda i: (0, 0))],
    out_specs=pl.BlockSpec((1, CNT_BINS), lambda i: (0, 0)),
    out_shape=jax.ShapeDtypeStruct((1, CNT_BINS), jnp.float32),
)


def _inv_counts(dst, zcnt):
    return _invsum(_hist32(dst, zcnt)).reshape(-1)[:N].reshape(N, 1)


R = 2000
GRID = N // R
_rowspec = pl.BlockSpec((R, D), lambda i: (i, 0))
_wspec = pl.BlockSpec((D, D), lambda i: (0, 0))
_bspec = pl.BlockSpec((1, D), lambda i: (0, 0))
_vspec = pl.BlockSpec((R, 1), lambda i: (i, 0))
_row_out = jax.ShapeDtypeStruct((N, D), jnp.float32)


def _dot_t(x, w):
    return lax.dot_general(x, w, (((1,), (1,)), ((), ())),
                           preferred_element_type=jnp.float32,
                           precision=lax.Precision.HIGHEST)


def _dense1_body(xp, xa, wrp, wrc, wra, wrw, brp, bra, p0, yp, a1, ya):
    x = xp[...]
    p0[...] = _dot_t(x, wrp[...]) + brp[...]
    yp[...] = _dot_t(x, wrc[...])
    a = xa[...]
    a1[...] = _dot_t(a, wra[...]) + bra[...]
    ya[...] = _dot_t(a, wrw[...])


_dense1 = pl.pallas_call(
    _dense1_body,
    grid=(GRID,),
    in_specs=[_rowspec, _rowspec, _wspec, _wspec, _wspec, _wspec, _bspec, _bspec],
    out_specs=[_rowspec] * 4,
    out_shape=[_row_out] * 4,
)


def _dense2_body(p0, sw, sc, iw, ic, a1, w2p, w2c, w2w, w2a, b2p, b2a,
                 p02, yp2, ya2, outa):
    hp = jnp.maximum(p0[...] + sw[...] * iw[...] + sc[...] * ic[...], 0.0)
    ha = jnp.maximum(a1[...], 0.0)
    p02[...] = _dot_t(hp, w2p[...]) + b2p[...]
    yp2[...] = _dot_t(hp, w2c[...])
    ya2[...] = _dot_t(ha, w2w[...])
    outa[...] = _dot_t(ha, w2a[...]) + b2a[...]


_dense2 = pl.pallas_call(
    _dense2_body,
    grid=(GRID,),
    in_specs=[_rowspec, _rowspec, _rowspec, _vspec, _vspec, _rowspec,
              _wspec, _wspec, _wspec, _wspec, _bspec, _bspec],
    out_specs=[_rowspec] * 4,
    out_shape=[_row_out] * 4,
)


def _final_body(p02, sw, sc, iw, ic, outp):
    outp[...] = p02[...] + sw[...] * iw[...] + sc[...] * ic[...]


_final = pl.pallas_call(
    _final_body,
    grid=(GRID,),
    in_specs=[_rowspec, _rowspec, _rowspec, _vspec, _vspec],
    out_specs=_rowspec,
    out_shape=_row_out,
)


def kernel(x_paper, author_emb, edge_index_writes, edge_index_cites,
           W_rel1_writes, W_rel1_cites, W_root1_paper, W_root1_author,
           W_rel2_writes, W_rel2_cites, W_root2_paper, W_root2_author,
           b_root1_paper, b_root1_author, b_root2_paper, b_root2_author):
    padn = E_PAD - E
    pad_src = jnp.zeros((padn,), jnp.int32)
    pad_dst = jnp.full((padn,), PAD_DST, jnp.int32)
    src_w = jnp.concatenate([edge_index_writes[0], pad_src])
    dst_w = jnp.concatenate([edge_index_writes[1], pad_dst])
    src_c = jnp.concatenate([edge_index_cites[0], pad_src])
    dst_c = jnp.concatenate([edge_index_cites[1], pad_dst])
    zrows = jnp.zeros((STRIPE, D), jnp.float32)
    zcnt = jnp.zeros((CNT_BINS,), jnp.float32)

    iw = _inv_counts(dst_w, zcnt)
    ic = _inv_counts(dst_c, zcnt)

    p0, yp, a1, ya = _dense1(x_paper, author_emb,
                             W_root1_paper, W_rel1_cites,
                             W_root1_author, W_rel1_writes,
                             b_root1_paper.reshape(1, D),
                             b_root1_author.reshape(1, D))
    sw1 = _segsum(ya, src_w, dst_w, zrows)
    sc1 = _segsum(yp, src_c, dst_c, zrows)
    p02, yp2, ya2, out_a = _dense2(p0, sw1, sc1, iw, ic, a1,
                                   W_root2_paper, W_rel2_cites,
                                   W_rel2_writes, W_root2_author,
                                   b_root2_paper.reshape(1, D),
                                   b_root2_author.reshape(1, D))
    sw2 = _segsum(ya2, src_w, dst_w, zrows)
    sc2 = _segsum(yp2, src_c, dst_c, zrows)
    out_p = _final(p02, sw2, sc2, iw, ic)
    return (out_p, out_a)


# speedup vs baseline: 27.2653x; 27.2653x over previous
import functools

import jax
import jax.numpy as jnp
from jax import lax
from jax.experimental import pallas as pl
from jax.experimental.pallas import tpu as pltpu
from jax.experimental.pallas import tpu_sc as plsc

N = 50000
D = 128
E = 300000

NT = 16
CH = 128
CHUNKS = 152
EPT = CH * CHUNKS
E_PAD = EPT * NT
SEG = 6272
NPASS = 4
TRASH = SEG
ACC_ROWS = 6400
STRIPE = ACC_ROWS // NT
PAD_DST = 50168

NSEG = 8
CAP = 24448
E2 = 32 * CAP

CNT_BINS = 53248
EPT32 = E_PAD // 32


_SEG_SCRATCH = (
    [pltpu.VMEM((CAP,), jnp.int32)] * 2
    + [pltpu.VMEM((128,), jnp.int32)]
    + [pltpu.VMEM((CH, D), jnp.float32)]
    + [pltpu.VMEM((CH,), jnp.int32)]
    + [pltpu.VMEM_SHARED((ACC_ROWS, D), jnp.float32)]
    + [pltpu.SemaphoreType.DMA] * 2
)


@functools.partial(
    pl.kernel,
    out_type=jax.ShapeDtypeStruct((N, D), jnp.float32),
    mesh=plsc.VectorSubcoreMesh(core_axis_name="c", subcore_axis_name="s"),
    scratch_types=_SEG_SCRATCH,
)
def _segsum(table, src2, dst2, runs, zrows, out,
            srcv, dstv, runsv, rows, offs, acc, gsem, ssem):
    c = lax.axis_index("c")
    t = lax.axis_index("s")
    wid = c * NT + t

    pltpu.sync_copy(src2.at[pl.ds(wid * CAP, CAP)], srcv)
    pltpu.sync_copy(dst2.at[pl.ds(wid * CAP, CAP)], dstv)
    pltpu.sync_copy(runs.at[wid], runsv)
    rv = runsv[pl.ds(0, 16)]

    for q in range(NPASS):
        k = NPASS * c + q
        lo = k * SEG
        hi = jnp.minimum(lo + SEG, N)
        start = jnp.where(c == 0, rv[q], rv[NPASS + q])
        nch = jnp.where(c == 0, rv[8 + q], rv[8 + NPASS + q])

        pltpu.sync_copy(zrows, acc.at[pl.ds(t * STRIPE, STRIPE)])
        plsc.subcore_barrier()

        def step(ci, carry, lo=lo, hi=hi, start=start):
            e0 = pl.multiple_of((start + ci) * CH, CH)
            pltpu.async_copy(table.at[srcv.at[pl.ds(e0, CH)]], rows, gsem).wait()
            for g in range(CH // 16):
                d = dstv[pl.ds(e0 + g * 16, 16)]
                offs[pl.ds(g * 16, 16)] = jnp.where(d < hi, d - lo, TRASH)
            pltpu.async_copy(rows, acc.at[offs], ssem, add=True).wait()
            return carry

        lax.fori_loop(0, nch, step, 0)

        plsc.subcore_barrier()
        wb = pl.multiple_of(jnp.minimum(t * STRIPE, hi - lo - STRIPE), 8)
        pltpu.sync_copy(acc.at[pl.ds(wb, STRIPE)],
                        out.at[pl.ds(pl.multiple_of(lo + wb, 8), STRIPE)])
        plsc.subcore_barrier()


_REO_SCRATCH = [
    pltpu.VMEM((EPT,), jnp.int32),
    pltpu.VMEM((EPT,), jnp.int32),
    pltpu.VMEM((CAP,), jnp.int32),
    pltpu.VMEM((CAP,), jnp.int32),
    pltpu.VMEM((128,), jnp.int32),
    pltpu.SMEM((256,), jnp.int32),
]


@functools.partial(
    pl.kernel,
    out_type=(jax.ShapeDtypeStruct((E2,), jnp.int32),
              jax.ShapeDtypeStruct((E2,), jnp.int32),
              jax.ShapeDtypeStruct((32, 128), jnp.int32)),
    mesh=plsc.VectorSubcoreMesh(core_axis_name="c", subcore_axis_name="s"),
    scratch_types=_REO_SCRATCH,
)
def _reorder(src, dst, src2, dst2, runs, srcv, dstv, src2v, dst2v, runsv, cur):
    c = lax.axis_index("c")
    t = lax.axis_index("s")
    wid = c * NT + t
    pltpu.sync_copy(src.at[pl.ds(t * EPT, EPT)], srcv)
    pltpu.sync_copy(dst.at[pl.ds(t * EPT, EPT)], dstv)

    for i in range(128):
        cur[i] = 0

    def cnt_step(g, carry):
        d = dstv[pl.ds(g * 16, 16)]
        for l in range(16):
            seg = d[l] // SEG
            cur[l * NSEG + seg] = cur[l * NSEG + seg] + 1
        return carry

    lax.fori_loop(0, EPT // 16, cnt_step, 0)

    pos = 0
    seg_start, seg_nch, gaps = [], [], []
    for k in range(NSEG):
        s0 = pos
        for l in range(16):
            cnt_lk = cur[l * NSEG + k]
            cur[128 + l * NSEG + k] = pos
            pos = pos + ((cnt_lk + 31) // 16) * 16
        seg_len = pos - s0
        padded = ((seg_len + 127) // 128) * 128
        gaps.append((pos, s0 + padded - pos))
        pos = s0 + padded
        seg_start.append(s0)
        seg_nch.append(padded // 128)

    for k in range(NSEG):
        runsv[pl.ds(k, 16)] = jnp.broadcast_to(seg_start[k] // CH, (16,))
    for k in range(NSEG):
        runsv[pl.ds(NSEG + k, 16)] = jnp.broadcast_to(seg_nch[k], (16,))

    def app_step(g, carry):
        d = dstv[pl.ds(g * 16, 16)]
        s = srcv[pl.ds(g * 16, 16)]
        for l in range(16):
            dl = d[l]
            seg = dl // SEG
            cu = cur[128 + l * NSEG + seg]
            dst2v[pl.ds(cu, 16)] = jnp.broadcast_to(dl, (16,))
            src2v[pl.ds(cu, 16)] = jnp.broadcast_to(s[l], (16,))
            cur[128 + l * NSEG + seg] = cu + 1
        return carry

    lax.fori_loop(0, EPT // 16, app_step, 0)

    def fill(lo_end, count):
        def pad_step(j, carry, lo_end=lo_end):
            p = lo_end - 16 - j
            dst2v[pl.ds(p, 16)] = jnp.broadcast_to(PAD_DST, (16,))
            src2v[pl.ds(p, 16)] = jnp.broadcast_to(0, (16,))
            return carry
        lax.fori_loop(0, count, pad_step, 0)

    pos2 = 0
    for k in range(NSEG):
        for l in range(16):
            cnt_lk = cur[l * NSEG + k]
            sub_end = pos2 + ((cnt_lk + 31) // 16) * 16
            fill(sub_end, sub_end - 15 - cur[128 + l * NSEG + k])
            pos2 = sub_end
        gap_pos, gap_len = gaps[k]
        fill(gap_pos + gap_len, gap_len)
        pos2 = gap_pos + gap_len

    pltpu.sync_copy(src2v, src2.at[pl.ds(wid * CAP, CAP)])
    pltpu.sync_copy(dst2v, dst2.at[pl.ds(wid * CAP, CAP)])
    pltpu.sync_copy(runsv, runs.at[wid])


_CNT_SCRATCH = [
    pltpu.VMEM((CNT_BINS,), jnp.float32),
    pltpu.VMEM((EPT32,), jnp.int32),
]


@functools.partial(
    pl.kernel,
    out_type=jax.ShapeDtypeStruct((32, CNT_BINS), jnp.float32),
    mesh=plsc.VectorSubcoreMesh(core_axis_name="c", subcore_axis_name="s"),
    scratch_types=_CNT_SCRATCH,
)
def _hist32(dst, zcnt, out, hist, dstv):
    c = lax.axis_index("c")
    t = lax.axis_index("s")
    wid = c * NT + t
    pltpu.sync_copy(zcnt, hist)
    pltpu.sync_copy(dst.at[pl.ds(wid * EPT32, EPT32)], dstv)

    inc = jnp.where(lax.iota(jnp.int32, 16) == 0, 1.0, 0.0).astype(jnp.float32)

    def step(g, carry):
        dvec = dstv[pl.ds(g * 16, 16)]
        for l in range(16):
            dl = dvec[l]
            hist[pl.ds(dl, 16)] = hist[pl.ds(dl, 16)] + inc
        return carry

    lax.fori_loop(0, EPT32 // 16, step, 0)
    pltpu.sync_copy(hist, out.at[wid])


def _invsum_body(h, inv):
    inv[...] = 1.0 / jnp.maximum(jnp.sum(h[...], axis=0), 1.0)[None, :]


_invsum = pl.pallas_call(
    _invsum_body,
    grid=(1,),
    in_specs=[pl.BlockSpec((32, CNT_BINS), lambda i: (0, 0))],
    out_specs=pl.BlockSpec((1, CNT_BINS), lambda i: (0, 0)),
    out_shape=jax.ShapeDtypeStruct((1, CNT_BINS), jnp.float32),
)


def _inv_counts(dst, zcnt):
    return _invsum(_hist32(dst, zcnt)).reshape(-1)[:N].reshape(N, 1)


R = 2000
GRID = N // R
_rowspec = pl.BlockSpec((R, D), lambda i: (i, 0))
_wspec = pl.BlockSpec((D, D), lambda i: (0, 0))
_bspec = pl.BlockSpec((1, D), lambda i: (0, 0))
_vspec = pl.BlockSpec((R, 1), lambda i: (i, 0))
_row_out = jax.ShapeDtypeStruct((N, D), jnp.float32)


def _dot_t(x, w):
    return lax.dot_general(x, w, (((1,), (1,)), ((), ())),
                           preferred_element_type=jnp.float32,
                           precision=lax.Precision.HIGHEST)


def _dense1_body(xp, xa, wrp, wrc, wra, wrw, brp, bra, p0, yp, a1, ya):
    x = xp[...]
    p0[...] = _dot_t(x, wrp[...]) + brp[...]
    yp[...] = _dot_t(x, wrc[...])
    a = xa[...]
    a1[...] = _dot_t(a, wra[...]) + bra[...]
    ya[...] = _dot_t(a, wrw[...])


_dense1 = pl.pallas_call(
    _dense1_body,
    grid=(GRID,),
    in_specs=[_rowspec, _rowspec, _wspec, _wspec, _wspec, _wspec, _bspec, _bspec],
    out_specs=[_rowspec] * 4,
    out_shape=[_row_out] * 4,
)


def _dense2_body(p0, sw, sc, iw, ic, a1, w2p, w2c, w2w, w2a, b2p, b2a,
                 p02, yp2, ya2, outa):
    hp = jnp.maximum(p0[...] + sw[...] * iw[...] + sc[...] * ic[...], 0.0)
    ha = jnp.maximum(a1[...], 0.0)
    p02[...] = _dot_t(hp, w2p[...]) + b2p[...]
    yp2[...] = _dot_t(hp, w2c[...])
    ya2[...] = _dot_t(ha, w2w[...])
    outa[...] = _dot_t(ha, w2a[...]) + b2a[...]


_dense2 = pl.pallas_call(
    _dense2_body,
    grid=(GRID,),
    in_specs=[_rowspec, _rowspec, _rowspec, _vspec, _vspec, _rowspec,
              _wspec, _wspec, _wspec, _wspec, _bspec, _bspec],
    out_specs=[_rowspec] * 4,
    out_shape=[_row_out] * 4,
)


def _final_body(p02, sw, sc, iw, ic, outp):
    outp[...] = p02[...] + sw[...] * iw[...] + sc[...] * ic[...]


_final = pl.pallas_call(
    _final_body,
    grid=(GRID,),
    in_specs=[_rowspec, _rowspec, _rowspec, _vspec, _vspec],
    out_specs=_rowspec,
    out_shape=_row_out,
)


def kernel(x_paper, author_emb, edge_index_writes, edge_index_cites,
           W_rel1_writes, W_rel1_cites, W_root1_paper, W_root1_author,
           W_rel2_writes, W_rel2_cites, W_root2_paper, W_root2_author,
           b_root1_paper, b_root1_author, b_root2_paper, b_root2_author):
    padn = E_PAD - E
    pad_src = jnp.zeros((padn,), jnp.int32)
    pad_dst = jnp.full((padn,), PAD_DST, jnp.int32)
    src_w = jnp.concatenate([edge_index_writes[0], pad_src])
    dst_w = jnp.concatenate([edge_index_writes[1], pad_dst])
    src_c = jnp.concatenate([edge_index_cites[0], pad_src])
    dst_c = jnp.concatenate([edge_index_cites[1], pad_dst])
    zrows = jnp.zeros((STRIPE, D), jnp.float32)
    zcnt = jnp.zeros((CNT_BINS,), jnp.float32)

    iw = _inv_counts(dst_w, zcnt)
    ic = _inv_counts(dst_c, zcnt)

    src2_w, dst2_w, runs_w = _reorder(src_w, dst_w)
    src2_c, dst2_c, runs_c = _reorder(src_c, dst_c)

    p0, yp, a1, ya = _dense1(x_paper, author_emb,
                             W_root1_paper, W_rel1_cites,
                             W_root1_author, W_rel1_writes,
                             b_root1_paper.reshape(1, D),
                             b_root1_author.reshape(1, D))
    sw1 = _segsum(ya, src2_w, dst2_w, runs_w, zrows)
    sc1 = _segsum(yp, src2_c, dst2_c, runs_c, zrows)
    p02, yp2, ya2, out_a = _dense2(p0, sw1, sc1, iw, ic, a1,
                                   W_root2_paper, W_rel2_cites,
                                   W_rel2_writes, W_root2_author,
                                   b_root2_paper.reshape(1, D),
                                   b_root2_author.reshape(1, D))
    sw2 = _segsum(ya2, src2_w, dst2_w, runs_w, zrows)
    sc2 = _segsum(yp2, src2_c, dst2_c, runs_c, zrows)
    out_p = _final(p02, sw2, sc2, iw, ic)
    return (out_p, out_a)
